# baseline (speedup 1.0000x reference)
import functools

import jax
import jax.numpy as jnp
from jax import lax
from jax.experimental import pallas as pl
from jax.experimental.pallas import tpu as pltpu
from jax.experimental.pallas import tpu_sc as plsc

VOCAB = 1000
EMBD = 64
L = 16
NC, NS = 2, 16
NW = NC * NS
W_TOTAL = 1024 * 200
WPW = W_TOTAL // NW
CHUNK = 256
NCHUNK = WPW // CHUNK
GROUPS = CHUNK // L
DBLK = EMBD // L


def _sc_char_embed(x_hbm, emb_hbm, out_hbm, tab_v, idx_v, out_v):
    wid = lax.axis_index("s") * NC + lax.axis_index("c")
    pltpu.sync_copy(emb_hbm, tab_v)

    iota = lax.iota(jnp.int32, L)
    i16 = iota * 16
    i64 = iota * EMBD
    base_w = wid * WPW

    def chunk_body(c, carry):
        w0 = base_w + c * CHUNK
        pltpu.sync_copy(x_hbm.at[pl.ds(w0 * 16, CHUNK * 16)], idx_v)

        def group_body(g, carry_g):
            gbase = g * (L * 16)
            xj64 = [
                plsc.load_gather(idx_v, [i16 + (gbase + j)]) * EMBD
                for j in range(16)
            ]
            obase = g * (L * EMBD)
            for dim in range(EMBD):
                vals = [
                    plsc.load_gather(tab_v, [xj64[j] + dim])
                    for j in range(16)
                ]
                while len(vals) > 1:
                    vals = [
                        vals[i] + vals[i + 1]
                        for i in range(0, len(vals), 2)
                    ]
                plsc.store_scatter(out_v, [i64 + (obase + dim)], vals[0])
            return carry_g

        lax.fori_loop(0, GROUPS, group_body, 0)
        pltpu.sync_copy(out_v, out_hbm.at[pl.ds(w0 * EMBD, CHUNK * EMBD)])
        return carry

    lax.fori_loop(0, NCHUNK, chunk_body, 0)


@jax.jit
def _char_embed_sc(x_flat, emb_flat):
    mesh = plsc.VectorSubcoreMesh(core_axis_name="c", subcore_axis_name="s")
    run = pl.kernel(
        _sc_char_embed,
        out_type=jax.ShapeDtypeStruct((W_TOTAL * EMBD,), jnp.float32),
        mesh=mesh,
        scratch_types=[
            pltpu.VMEM((VOCAB * EMBD,), jnp.float32),
            pltpu.VMEM((CHUNK * 16,), jnp.int32),
            pltpu.VMEM((CHUNK * EMBD,), jnp.float32),
        ],
        compiler_params=pltpu.CompilerParams(needs_layout_passes=False),
    )
    return run(x_flat, emb_flat)


def kernel(x, emb):
    bs, seq, word = x.shape
    out = _char_embed_sc(
        x.reshape(-1).astype(jnp.int32),
        emb.reshape(-1),
    )
    return out.reshape(bs, seq, EMBD)


# speedup vs baseline: 1.1307x; 1.1307x over previous
import functools

import jax
import jax.numpy as jnp
from jax import lax
from jax.experimental import pallas as pl
from jax.experimental.pallas import tpu as pltpu
from jax.experimental.pallas import tpu_sc as plsc

VOCAB = 1000
EMBD = 64
L = 16
NC, NS = 2, 16
NW = NC * NS
W_TOTAL = 1024 * 200
WPW = W_TOTAL // NW
CHUNK = 256
NCHUNK = WPW // CHUNK
GROUPS = CHUNK // L
DBLK = EMBD // L


def _sc_char_embed(x_hbm, emb_hbm, out_hbm, tab_v, idx_v, out_v):
    wid = lax.axis_index("s") * NC + lax.axis_index("c")
    pltpu.sync_copy(emb_hbm, tab_v)

    iota = lax.iota(jnp.int32, L)
    i16 = iota * 16
    i64 = iota * EMBD
    base_w = wid * WPW

    def chunk_body(c, carry):
        w0 = base_w + c * CHUNK
        pltpu.sync_copy(x_hbm.at[pl.ds(w0 * 16, CHUNK * 16)], idx_v)

        def group_body(g, carry_g):
            gbase = g * (L * 16)
            xj64 = [
                plsc.load_gather(idx_v, [i16 + (gbase + j)]) * EMBD
                for j in range(16)
            ]
            obase = g * (L * EMBD)

            @plsc.parallel_loop(0, EMBD, unroll=4)
            def dim_loop(dim):
                vals = [
                    plsc.load_gather(tab_v, [xj64[j] + dim])
                    for j in range(16)
                ]
                while len(vals) > 1:
                    vals = [
                        vals[i] + vals[i + 1]
                        for i in range(0, len(vals), 2)
                    ]
                plsc.store_scatter(out_v, [i64 + (obase + dim)], vals[0])

            return carry_g

        lax.fori_loop(0, GROUPS, group_body, 0)
        pltpu.sync_copy(out_v, out_hbm.at[pl.ds(w0 * EMBD, CHUNK * EMBD)])
        return carry

    lax.fori_loop(0, NCHUNK, chunk_body, 0)


@jax.jit
def _char_embed_sc(x_flat, emb_flat):
    mesh = plsc.VectorSubcoreMesh(core_axis_name="c", subcore_axis_name="s")
    run = pl.kernel(
        _sc_char_embed,
        out_type=jax.ShapeDtypeStruct((W_TOTAL * EMBD,), jnp.float32),
        mesh=mesh,
        scratch_types=[
            pltpu.VMEM((VOCAB * EMBD,), jnp.float32),
            pltpu.VMEM((CHUNK * 16,), jnp.int32),
            pltpu.VMEM((CHUNK * EMBD,), jnp.float32),
        ],
        compiler_params=pltpu.CompilerParams(needs_layout_passes=False),
    )
    return run(x_flat, emb_flat)


def kernel(x, emb):
    bs, seq, word = x.shape
    out = _char_embed_sc(
        x.reshape(-1).astype(jnp.int32),
        emb.reshape(-1),
    )
    return out.reshape(bs, seq, EMBD)


# speedup vs baseline: 7.3589x; 6.5086x over previous
import functools

import jax
import jax.numpy as jnp
from jax import lax
from jax.experimental import pallas as pl
from jax.experimental.pallas import tpu as pltpu
from jax.experimental.pallas import tpu_sc as plsc

VOCAB = 1000
VPAD = 1024
EMBD = 64
KCOL = EMBD // 2
L = 16
NC, NS = 2, 16
NW = NC * NS
W_TOTAL = 1024 * 200
WPW = W_TOTAL // NW
CHUNK = 256
NCHUNK = WPW // CHUNK
GROUPS = CHUNK // L


def _sc_char_embed(x_hbm, tab_hbm, out_hbm, tab_v, idx_v, out_v):
    wid = lax.axis_index("s") * NC + lax.axis_index("c")
    pltpu.sync_copy(tab_hbm, tab_v)

    iota = lax.iota(jnp.int32, L)
    i16 = iota * 16
    i64 = iota * EMBD
    base_w = wid * WPW

    def chunk_body(c, carry):
        w0 = base_w + c * CHUNK
        pltpu.sync_copy(x_hbm.at[pl.ds(w0 * 16, CHUNK * 16)], idx_v)

        def group_body(g, carry_g):
            gbase = g * (L * 16)
            xjs = [
                plsc.load_gather(idx_v, [i16 + (gbase + j)])
                for j in range(16)
            ]
            obase = g * (L * EMBD)

            @plsc.parallel_loop(0, KCOL, unroll=2)
            def k_loop(k):
                kb = k * VPAD
                vals = [
                    plsc.bitcast(
                        plsc.load_gather(tab_v, [xjs[j] + kb]),
                        jnp.bfloat16,
                    )
                    for j in range(16)
                ]
                while len(vals) > 1:
                    vals = [
                        vals[i] + vals[i + 1]
                        for i in range(0, len(vals), 2)
                    ]
                a, b = plsc.unpack(vals[0], format=plsc.PackFormat.INTERLEAVED)
                plsc.store_scatter(out_v, [i64 + (obase + 2 * k)], a)
                plsc.store_scatter(out_v, [i64 + (obase + 2 * k + 1)], b)

            return carry_g

        lax.fori_loop(0, GROUPS, group_body, 0)
        pltpu.sync_copy(out_v, out_hbm.at[pl.ds(w0 * EMBD, CHUNK * EMBD)])
        return carry

    lax.fori_loop(0, NCHUNK, chunk_body, 0)


@jax.jit
def _char_embed_sc(x_flat, tab_flat):
    mesh = plsc.VectorSubcoreMesh(core_axis_name="c", subcore_axis_name="s")
    run = pl.kernel(
        _sc_char_embed,
        out_type=jax.ShapeDtypeStruct((W_TOTAL * EMBD,), jnp.float32),
        mesh=mesh,
        scratch_types=[
            pltpu.VMEM((KCOL * VPAD,), jnp.int32),
            pltpu.VMEM((CHUNK * 16,), jnp.int32),
            pltpu.VMEM((CHUNK * EMBD,), jnp.float32),
        ],
        compiler_params=pltpu.CompilerParams(needs_layout_passes=False),
    )
    return run(x_flat, tab_flat)


def _pack_table(emb):
    u16 = jax.lax.bitcast_convert_type(
        emb.astype(jnp.bfloat16).reshape(VOCAB, KCOL, 2), jnp.uint16
    ).astype(jnp.uint32)
    u32 = u16[:, :, 0] | (u16[:, :, 1] << 16)
    padded = jnp.zeros((VPAD, KCOL), jnp.uint32).at[:VOCAB].set(u32)
    return jax.lax.bitcast_convert_type(padded.T.reshape(-1), jnp.int32)


def kernel(x, emb):
    bs, seq, word = x.shape
    out = _char_embed_sc(
        x.reshape(-1).astype(jnp.int32),
        _pack_table(emb),
    )
    return out.reshape(bs, seq, EMBD)


# speedup vs baseline: 7.8580x; 1.0678x over previous
import functools

import jax
import jax.numpy as jnp
from jax import lax
from jax.experimental import pallas as pl
from jax.experimental.pallas import tpu as pltpu
from jax.experimental.pallas import tpu_sc as plsc

VOCAB = 1000
VPAD = 1024
EMBD = 64
KCOL = EMBD // 2
L = 16
NC, NS = 2, 16
NW = NC * NS
W_TOTAL = 1024 * 200
WPW = W_TOTAL // NW
CHUNK = 400
NCHUNK = WPW // CHUNK
NPAIR = NCHUNK // 2
GROUPS = CHUNK // L


def _sc_char_embed(x_hbm, tab_hbm, out_hbm, tab_v,
                   idx_a, idx_b, out_a, out_b, sia, sib, soa, sob):
    wid = lax.axis_index("s") * NC + lax.axis_index("c")
    pltpu.sync_copy(tab_hbm, tab_v)

    iota = lax.iota(jnp.int32, L)
    i16 = iota * 16
    i64 = iota * EMBD
    base_w = wid * WPW

    def idx_slice(c):
        return x_hbm.at[pl.ds((base_w + c * CHUNK) * 16, CHUNK * 16)]

    def out_slice(c):
        return out_hbm.at[pl.ds((base_w + c * CHUNK) * EMBD, CHUNK * EMBD)]

    def compute(idx_v, out_v):
        def group_body(g, carry_g):
            gbase = g * (L * 16)
            xjs = [
                plsc.load_gather(idx_v, [i16 + (gbase + j)])
                for j in range(16)
            ]
            obase = g * (L * EMBD)

            @plsc.parallel_loop(0, KCOL, unroll=2)
            def k_loop(k):
                kb = k * VPAD
                vals = [
                    plsc.bitcast(
                        plsc.load_gather(tab_v, [xjs[j] + kb]),
                        jnp.bfloat16,
                    )
                    for j in range(16)
                ]
                while len(vals) > 1:
                    vals = [
                        vals[i] + vals[i + 1]
                        for i in range(0, len(vals), 2)
                    ]
                a, b = plsc.unpack(vals[0], format=plsc.PackFormat.INTERLEAVED)
                plsc.store_scatter(out_v, [i64 + (obase + 2 * k)], a)
                plsc.store_scatter(out_v, [i64 + (obase + 2 * k + 1)], b)

            return carry_g

        lax.fori_loop(0, GROUPS, group_body, 0)

    pltpu.async_copy(idx_slice(0), idx_a, sia)

    def pair_body(t, carry):
        c_a = 2 * t
        c_b = 2 * t + 1
        pltpu.async_copy(idx_slice(c_b), idx_b, sib)
        pltpu.make_async_copy(idx_slice(c_a), idx_a, sia).wait()

        @pl.when(t > 0)
        def _wait_out_a():
            pltpu.make_async_copy(out_a, out_slice(c_a - 2), soa).wait()

        compute(idx_a, out_a)
        pltpu.async_copy(out_a, out_slice(c_a), soa)

        @pl.when(t < NPAIR - 1)
        def _prefetch_a():
            pltpu.async_copy(idx_slice(c_a + 2), idx_a, sia)

        pltpu.make_async_copy(idx_slice(c_b), idx_b, sib).wait()

        @pl.when(t > 0)
        def _wait_out_b():
            pltpu.make_async_copy(out_b, out_slice(c_b - 2), sob).wait()

        compute(idx_b, out_b)
        pltpu.async_copy(out_b, out_slice(c_b), sob)
        return carry

    lax.fori_loop(0, NPAIR, pair_body, 0)
    pltpu.make_async_copy(out_a, out_slice(NCHUNK - 2), soa).wait()
    pltpu.make_async_copy(out_b, out_slice(NCHUNK - 1), sob).wait()


@jax.jit
def _char_embed_sc(x_flat, tab_flat):
    mesh = plsc.VectorSubcoreMesh(core_axis_name="c", subcore_axis_name="s")
    run = pl.kernel(
        _sc_char_embed,
        out_type=jax.ShapeDtypeStruct((W_TOTAL * EMBD,), jnp.float32),
        mesh=mesh,
        scratch_types=[
            pltpu.VMEM((KCOL * VPAD,), jnp.int32),
            pltpu.VMEM((CHUNK * 16,), jnp.int32),
            pltpu.VMEM((CHUNK * 16,), jnp.int32),
            pltpu.VMEM((CHUNK * EMBD,), jnp.float32),
            pltpu.VMEM((CHUNK * EMBD,), jnp.float32),
            pltpu.SemaphoreType.DMA,
            pltpu.SemaphoreType.DMA,
            pltpu.SemaphoreType.DMA,
            pltpu.SemaphoreType.DMA,
        ],
        compiler_params=pltpu.CompilerParams(needs_layout_passes=False),
    )
    return run(x_flat, tab_flat)


def _pack_table(emb):
    u16 = jax.lax.bitcast_convert_type(
        emb.astype(jnp.bfloat16).reshape(VOCAB, KCOL, 2), jnp.uint16
    ).astype(jnp.uint32)
    u32 = u16[:, :, 0] | (u16[:, :, 1] << 16)
    padded = jnp.zeros((VPAD, KCOL), jnp.uint32).at[:VOCAB].set(u32)
    return jax.lax.bitcast_convert_type(padded.T.reshape(-1), jnp.int32)


def kernel(x, emb):
    bs, seq, word = x.shape
    out = _char_embed_sc(
        x.reshape(-1).astype(jnp.int32),
        _pack_table(emb),
    )
    return out.reshape(bs, seq, EMBD)


# speedup vs baseline: 7.8644x; 1.0008x over previous
import functools

import jax
import jax.numpy as jnp
from jax import lax
from jax.experimental import pallas as pl
from jax.experimental.pallas import tpu as pltpu
from jax.experimental.pallas import tpu_sc as plsc

VOCAB = 1000
VPAD = 1024
EMBD = 64
KCOL = EMBD // 2
L = 16
NC, NS = 2, 16
NW = NC * NS
W_TOTAL = 1024 * 200
WPW = W_TOTAL // NW
CHUNK = 400
NCHUNK = WPW // CHUNK
NPAIR = NCHUNK // 2
GROUPS = CHUNK // L


def _sc_char_embed(x_hbm, tab_hbm, out_hbm, tab_v,
                   idx_a, idx_b, out_a, out_b, sia, sib, soa, sob):
    wid = lax.axis_index("s") * NC + lax.axis_index("c")
    pltpu.sync_copy(tab_hbm, tab_v)

    iota = lax.iota(jnp.int32, L)
    i16 = iota * 16
    i64 = iota * EMBD
    base_w = wid * WPW

    def idx_slice(c):
        return x_hbm.at[pl.ds((base_w + c * CHUNK) * 16, CHUNK * 16)]

    def out_slice(c):
        return out_hbm.at[pl.ds((base_w + c * CHUNK) * EMBD, CHUNK * EMBD)]

    def compute(idx_v, out_v):
        def group_body(g, carry_g):
            gbase = g * (L * 16)
            xjs = [
                plsc.load_gather(idx_v, [i16 + (gbase + j)])
                for j in range(16)
            ]
            obase = g * (L * EMBD)

            @plsc.parallel_loop(0, KCOL, unroll=2)
            def k_loop(k):
                col = tab_v.at[pl.ds(k * VPAD, VPAD)]
                vals = [
                    plsc.bitcast(
                        plsc.load_gather(col, [xjs[j]]),
                        jnp.bfloat16,
                    )
                    for j in range(16)
                ]
                while len(vals) > 1:
                    vals = [
                        vals[i] + vals[i + 1]
                        for i in range(0, len(vals), 2)
                    ]
                a, b = plsc.unpack(vals[0], format=plsc.PackFormat.INTERLEAVED)
                plsc.store_scatter(out_v, [i64 + (obase + 2 * k)], a)
                plsc.store_scatter(out_v, [i64 + (obase + 2 * k + 1)], b)

            return carry_g

        lax.fori_loop(0, GROUPS, group_body, 0)

    pltpu.async_copy(idx_slice(0), idx_a, sia)

    def pair_body(t, carry):
        c_a = 2 * t
        c_b = 2 * t + 1
        pltpu.async_copy(idx_slice(c_b), idx_b, sib)
        pltpu.make_async_copy(idx_slice(c_a), idx_a, sia).wait()

        @pl.when(t > 0)
        def _wait_out_a():
            pltpu.make_async_copy(out_a, out_slice(c_a - 2), soa).wait()

        compute(idx_a, out_a)
        pltpu.async_copy(out_a, out_slice(c_a), soa)

        @pl.when(t < NPAIR - 1)
        def _prefetch_a():
            pltpu.async_copy(idx_slice(c_a + 2), idx_a, sia)

        pltpu.make_async_copy(idx_slice(c_b), idx_b, sib).wait()

        @pl.when(t > 0)
        def _wait_out_b():
            pltpu.make_async_copy(out_b, out_slice(c_b - 2), sob).wait()

        compute(idx_b, out_b)
        pltpu.async_copy(out_b, out_slice(c_b), sob)
        return carry

    lax.fori_loop(0, NPAIR, pair_body, 0)
    pltpu.make_async_copy(out_a, out_slice(NCHUNK - 2), soa).wait()
    pltpu.make_async_copy(out_b, out_slice(NCHUNK - 1), sob).wait()


@jax.jit
def _char_embed_sc(x_flat, tab_flat):
    mesh = plsc.VectorSubcoreMesh(core_axis_name="c", subcore_axis_name="s")
    run = pl.kernel(
        _sc_char_embed,
        out_type=jax.ShapeDtypeStruct((W_TOTAL * EMBD,), jnp.float32),
        mesh=mesh,
        scratch_types=[
            pltpu.VMEM((KCOL * VPAD,), jnp.int32),
            pltpu.VMEM((CHUNK * 16,), jnp.int32),
            pltpu.VMEM((CHUNK * 16,), jnp.int32),
            pltpu.VMEM((CHUNK * EMBD,), jnp.float32),
            pltpu.VMEM((CHUNK * EMBD,), jnp.float32),
            pltpu.SemaphoreType.DMA,
            pltpu.SemaphoreType.DMA,
            pltpu.SemaphoreType.DMA,
            pltpu.SemaphoreType.DMA,
        ],
        compiler_params=pltpu.CompilerParams(needs_layout_passes=False),
    )
    return run(x_flat, tab_flat)


def _pack_table(emb):
    u16 = jax.lax.bitcast_convert_type(
        emb.astype(jnp.bfloat16).reshape(VOCAB, KCOL, 2), jnp.uint16
    ).astype(jnp.uint32)
    u32 = u16[:, :, 0] | (u16[:, :, 1] << 16)
    padded = jnp.zeros((VPAD, KCOL), jnp.uint32).at[:VOCAB].set(u32)
    return jax.lax.bitcast_convert_type(padded.T.reshape(-1), jnp.int32)


def kernel(x, emb):
    bs, seq, word = x.shape
    out = _char_embed_sc(
        x.reshape(-1).astype(jnp.int32),
        _pack_table(emb),
    )
    return out.reshape(bs, seq, EMBD)


# speedup vs baseline: 10.3175x; 1.3119x over previous
import functools

import jax
import jax.numpy as jnp
from jax import lax
from jax.experimental import pallas as pl
from jax.experimental.pallas import tpu as pltpu
from jax.experimental.pallas import tpu_sc as plsc

VOCAB = 1000
EMBD = 64
KCOL = EMBD // 2
L = 16
NC, NS = 2, 16
NW = NC * NS
W_TOTAL = 1024 * 200
WPW = W_TOTAL // NW
CHUNK = 400
NCHUNK = WPW // CHUNK
NPAIR = NCHUNK // 2


def _sc_char_embed(x_hbm, tab_hbm, out_hbm, tab_v,
                   idx_a, idx_b, out_a, out_b, sia, sib, soa, sob):
    wid = lax.axis_index("s") * NC + lax.axis_index("c")
    pltpu.sync_copy(tab_hbm, tab_v)

    iota = lax.iota(jnp.int32, L)
    ihi = iota + L
    splats = [jnp.full((L,), j, jnp.int32) for j in range(16)]
    base_w = wid * WPW

    def idx_slice(c):
        return x_hbm.at[pl.ds((base_w + c * CHUNK) * 16, CHUNK * 16)]

    def out_slice(c):
        return out_hbm.at[pl.ds((base_w + c * CHUNK) * EMBD, CHUNK * EMBD)]

    def compute(idx_v, out_v):
        @plsc.parallel_loop(0, CHUNK, unroll=2)
        def word_loop(w):
            cvec = idx_v[pl.ds(w * 16, L)] * KCOL
            acc0 = None
            acc1 = None
            for j in range(16):
                rowb = cvec.at[splats[j]].get(mode="promise_in_bounds")
                v0 = plsc.bitcast(
                    plsc.load_gather(tab_v, [rowb + iota]), jnp.bfloat16)
                v1 = plsc.bitcast(
                    plsc.load_gather(tab_v, [rowb + ihi]), jnp.bfloat16)
                acc0 = v0 if acc0 is None else acc0 + v0
                acc1 = v1 if acc1 is None else acc1 + v1
            a0, b0 = plsc.unpack(acc0, format=plsc.PackFormat.INTERLEAVED)
            a1, b1 = plsc.unpack(acc1, format=plsc.PackFormat.INTERLEAVED)
            ob = w * EMBD
            out_v[pl.ds(ob, L)] = a0
            out_v[pl.ds(ob + 16, L)] = a1
            out_v[pl.ds(ob + 32, L)] = b0
            out_v[pl.ds(ob + 48, L)] = b1

    pltpu.async_copy(idx_slice(0), idx_a, sia)

    def pair_body(t, carry):
        c_a = 2 * t
        c_b = 2 * t + 1
        pltpu.async_copy(idx_slice(c_b), idx_b, sib)
        pltpu.make_async_copy(idx_slice(c_a), idx_a, sia).wait()

        @pl.when(t > 0)
        def _wait_out_a():
            pltpu.make_async_copy(out_a, out_slice(c_a - 2), soa).wait()

        compute(idx_a, out_a)
        pltpu.async_copy(out_a, out_slice(c_a), soa)

        @pl.when(t < NPAIR - 1)
        def _prefetch_a():
            pltpu.async_copy(idx_slice(c_a + 2), idx_a, sia)

        pltpu.make_async_copy(idx_slice(c_b), idx_b, sib).wait()

        @pl.when(t > 0)
        def _wait_out_b():
            pltpu.make_async_copy(out_b, out_slice(c_b - 2), sob).wait()

        compute(idx_b, out_b)
        pltpu.async_copy(out_b, out_slice(c_b), sob)
        return carry

    lax.fori_loop(0, NPAIR, pair_body, 0)
    pltpu.make_async_copy(out_a, out_slice(NCHUNK - 2), soa).wait()
    pltpu.make_async_copy(out_b, out_slice(NCHUNK - 1), sob).wait()


@jax.jit
def _char_embed_sc(x_flat, tab_flat):
    mesh = plsc.VectorSubcoreMesh(core_axis_name="c", subcore_axis_name="s")
    run = pl.kernel(
        _sc_char_embed,
        out_type=jax.ShapeDtypeStruct((W_TOTAL * EMBD,), jnp.float32),
        mesh=mesh,
        scratch_types=[
            pltpu.VMEM((VOCAB * KCOL,), jnp.int32),
            pltpu.VMEM((CHUNK * 16,), jnp.int32),
            pltpu.VMEM((CHUNK * 16,), jnp.int32),
            pltpu.VMEM((CHUNK * EMBD,), jnp.float32),
            pltpu.VMEM((CHUNK * EMBD,), jnp.float32),
            pltpu.SemaphoreType.DMA,
            pltpu.SemaphoreType.DMA,
            pltpu.SemaphoreType.DMA,
            pltpu.SemaphoreType.DMA,
        ],
        compiler_params=pltpu.CompilerParams(needs_layout_passes=False),
    )
    return run(x_flat, tab_flat)


def _pack_table(emb):
    u16 = jax.lax.bitcast_convert_type(
        emb.astype(jnp.bfloat16), jnp.uint16
    ).astype(jnp.uint32)
    u32 = u16[:, :KCOL] | (u16[:, KCOL:] << 16)
    return jax.lax.bitcast_convert_type(u32.reshape(-1), jnp.int32)


def kernel(x, emb):
    bs, seq, word = x.shape
    out = _char_embed_sc(
        x.reshape(-1).astype(jnp.int32),
        _pack_table(emb),
    )
    return out.reshape(bs, seq, EMBD)


# speedup vs baseline: 12.3212x; 1.1942x over previous
import functools

import jax
import jax.numpy as jnp
from jax import lax
from jax.experimental import pallas as pl
from jax.experimental.pallas import tpu as pltpu
from jax.experimental.pallas import tpu_sc as plsc

VOCAB = 1000
EMBD = 64
KCOL = EMBD // 2
L = 16
NC, NS = 2, 16
NW = NC * NS
W_TOTAL = 1024 * 200
WPW = W_TOTAL // NW
CHUNK = 400
NCHUNK = WPW // CHUNK
NPAIR = NCHUNK // 2


def _sc_char_embed(x_hbm, tab_hbm, out_hbm, tab_v,
                   idx_a, idx_b, out_a, out_b, sia, sib, soa, sob):
    wid = lax.axis_index("s") * NC + lax.axis_index("c")
    pltpu.sync_copy(tab_hbm, tab_v)

    iota = lax.iota(jnp.int32, L)
    ihi = iota + L
    splats = [jnp.full((L,), j, jnp.int32) for j in range(16)]
    base_w = wid * WPW

    def idx_slice(c):
        return x_hbm.at[pl.ds((base_w + c * CHUNK) * 16, CHUNK * 16)]

    def out_slice(c):
        return out_hbm.at[pl.ds((base_w + c * CHUNK) * EMBD, CHUNK * EMBD)]

    def compute(idx_v, out_v):
        @plsc.parallel_loop(0, CHUNK, unroll=1)
        def word_loop(w):
            cvec = idx_v[pl.ds(w * 16, L)] * KCOL
            acc0 = None
            acc1 = None
            for j in range(16):
                rowb = cvec.at[splats[j]].get(mode="promise_in_bounds")
                v0 = plsc.bitcast(
                    plsc.load_gather(tab_v, [rowb + iota]), jnp.bfloat16)
                v1 = plsc.bitcast(
                    plsc.load_gather(tab_v, [rowb + ihi]), jnp.bfloat16)
                acc0 = v0 if acc0 is None else acc0 + v0
                acc1 = v1 if acc1 is None else acc1 + v1
            a0, b0 = plsc.unpack(acc0, format=plsc.PackFormat.INTERLEAVED)
            a1, b1 = plsc.unpack(acc1, format=plsc.PackFormat.INTERLEAVED)
            ob = w * EMBD
            out_v[pl.ds(ob, L)] = a0
            out_v[pl.ds(ob + 16, L)] = a1
            out_v[pl.ds(ob + 32, L)] = b0
            out_v[pl.ds(ob + 48, L)] = b1

    pltpu.async_copy(idx_slice(0), idx_a, sia)

    def pair_body(t, carry):
        c_a = 2 * t
        c_b = 2 * t + 1
        pltpu.async_copy(idx_slice(c_b), idx_b, sib)
        pltpu.make_async_copy(idx_slice(c_a), idx_a, sia).wait()

        @pl.when(t > 0)
        def _wait_out_a():
            pltpu.make_async_copy(out_a, out_slice(c_a - 2), soa).wait()

        compute(idx_a, out_a)
        pltpu.async_copy(out_a, out_slice(c_a), soa)

        @pl.when(t < NPAIR - 1)
        def _prefetch_a():
            pltpu.async_copy(idx_slice(c_a + 2), idx_a, sia)

        pltpu.make_async_copy(idx_slice(c_b), idx_b, sib).wait()

        @pl.when(t > 0)
        def _wait_out_b():
            pltpu.make_async_copy(out_b, out_slice(c_b - 2), sob).wait()

        compute(idx_b, out_b)
        pltpu.async_copy(out_b, out_slice(c_b), sob)
        return carry

    lax.fori_loop(0, NPAIR, pair_body, 0)
    pltpu.make_async_copy(out_a, out_slice(NCHUNK - 2), soa).wait()
    pltpu.make_async_copy(out_b, out_slice(NCHUNK - 1), sob).wait()


@jax.jit
def _char_embed_sc(x_flat, tab_flat):
    mesh = plsc.VectorSubcoreMesh(core_axis_name="c", subcore_axis_name="s")
    run = pl.kernel(
        _sc_char_embed,
        out_type=jax.ShapeDtypeStruct((W_TOTAL * EMBD,), jnp.float32),
        mesh=mesh,
        scratch_types=[
            pltpu.VMEM((VOCAB * KCOL,), jnp.int32),
            pltpu.VMEM((CHUNK * 16,), jnp.int32),
            pltpu.VMEM((CHUNK * 16,), jnp.int32),
            pltpu.VMEM((CHUNK * EMBD,), jnp.float32),
            pltpu.VMEM((CHUNK * EMBD,), jnp.float32),
            pltpu.SemaphoreType.DMA,
            pltpu.SemaphoreType.DMA,
            pltpu.SemaphoreType.DMA,
            pltpu.SemaphoreType.DMA,
        ],
        compiler_params=pltpu.CompilerParams(needs_layout_passes=False),
    )
    return run(x_flat, tab_flat)


def _pack_table(emb):
    u16 = jax.lax.bitcast_convert_type(
        emb.astype(jnp.bfloat16), jnp.uint16
    ).astype(jnp.uint32)
    u32 = u16[:, :KCOL] | (u16[:, KCOL:] << 16)
    return jax.lax.bitcast_convert_type(u32.reshape(-1), jnp.int32)


def kernel(x, emb):
    bs, seq, word = x.shape
    out = _char_embed_sc(
        x.reshape(-1).astype(jnp.int32),
        _pack_table(emb),
    )
    return out.reshape(bs, seq, EMBD)
